# Initial kernel scaffold; baseline (speedup 1.0000x reference)
import jax
import jax.numpy as jnp
from jax.experimental import pallas as pl


def kernel(num_prop, cat_prop, w_gate, W1, b1, W2, b2, Wo, bo):
    raise NotImplementedError("write your pallas kernel here")


# speedup vs baseline: 4.2800x; 4.2800x over previous
import jax
import jax.numpy as jnp
from jax.experimental import pallas as pl
from jax.experimental.pallas import tpu as pltpu

_N, _D, _E, _H, _M = 8192, 1024, 8, 128, 1024
_BN = 256
_GRID = _N // _BN


def _moe_body(x_ref, wg_ref, w1_ref, b1_ref, w2_ref, b2_ref, wo_ref, bo_ref,
              out_ref, loss_ref, imp_ref, load_ref):
    pid = pl.program_id(0)
    x = x_ref[...]
    logits = jnp.dot(x, wg_ref[...], preferred_element_type=jnp.float32)

    ids = jax.lax.broadcasted_iota(jnp.int32, (_BN, _E), 1)
    l1 = jnp.max(logits, axis=1, keepdims=True)
    i1 = jnp.min(jnp.where(logits == l1, ids, _E), axis=1, keepdims=True)
    masked = jnp.where(ids == i1, jnp.float32(-1e30), logits)
    l2 = jnp.max(masked, axis=1, keepdims=True)
    i2 = jnp.min(jnp.where(masked == l2, ids, _E), axis=1, keepdims=True)

    e21 = jnp.exp(l2 - l1)
    g1 = 1.0 / (1.0 + e21)
    g2 = e21 / (1.0 + e21)

    oh1 = (ids == i1).astype(jnp.float32)
    oh2 = (ids == i2).astype(jnp.float32)
    gates = oh1 * g1 + oh2 * g2

    @pl.when(pid == 0)
    def _():
        imp_ref[...] = jnp.zeros_like(imp_ref)
        load_ref[...] = jnp.zeros_like(load_ref)

    imp_ref[...] += jnp.sum(gates, axis=0, keepdims=True)
    load_ref[...] += jnp.sum((gates > 0).astype(jnp.float32), axis=0,
                             keepdims=True)

    h_all = jnp.maximum(
        jnp.dot(x, w1_ref[...], preferred_element_type=jnp.float32)
        + b1_ref[...], 0.0)

    acc = jnp.zeros((_BN, _M), dtype=jnp.float32)
    for e in range(_E):
        h = h_all[:, e * _H:(e + 1) * _H]
        z = jnp.dot(h, w2_ref[e], preferred_element_type=jnp.float32) \
            + b2_ref[e, :][None, :]
        mx = jnp.max(z, axis=1, keepdims=True)
        ez = jnp.exp(z - mx)
        s = jnp.sum(ez, axis=1, keepdims=True)
        ge = gates[:, e:e + 1]
        acc = acc + ez * (ge / s)

    out_ref[...] = jnp.dot(acc, wo_ref[...],
                           preferred_element_type=jnp.float32) + bo_ref[...]

    @pl.when(pid == _GRID - 1)
    def _():
        def cv2(v):
            m = jnp.sum(v) / _E
            d = v - m
            var = jnp.sum(d * d) / (_E - 1)
            return var / (m * m + 1e-10)
        loss = (cv2(imp_ref[...]) + cv2(load_ref[...])) * 1e-2
        loss_ref[...] = jnp.full((1, 1), loss, dtype=jnp.float32)


def kernel(num_prop, cat_prop, w_gate, W1, b1, W2, b2, Wo, bo):
    w1 = jnp.transpose(W1, (1, 0, 2)).reshape(_D, _E * _H)
    b1r = b1.reshape(1, _E * _H)
    bor = bo.reshape(1, 2)

    out, loss = pl.pallas_call(
        _moe_body,
        grid=(_GRID,),
        in_specs=[
            pl.BlockSpec((_BN, _D), lambda i: (i, 0)),
            pl.BlockSpec((_D, _E), lambda i: (0, 0)),
            pl.BlockSpec((_D, _E * _H), lambda i: (0, 0)),
            pl.BlockSpec((1, _E * _H), lambda i: (0, 0)),
            pl.BlockSpec((_E, _H, _M), lambda i: (0, 0, 0)),
            pl.BlockSpec((_E, _M), lambda i: (0, 0)),
            pl.BlockSpec((_M, 2), lambda i: (0, 0)),
            pl.BlockSpec((1, 2), lambda i: (0, 0)),
        ],
        out_specs=[
            pl.BlockSpec((_BN, 2), lambda i: (i, 0)),
            pl.BlockSpec((1, 1), lambda i: (0, 0)),
        ],
        out_shape=[
            jax.ShapeDtypeStruct((_N, 2), jnp.float32),
            jax.ShapeDtypeStruct((1, 1), jnp.float32),
        ],
        scratch_shapes=[
            pltpu.VMEM((1, _E), jnp.float32),
            pltpu.VMEM((1, _E), jnp.float32),
        ],
        compiler_params=pltpu.CompilerParams(
            dimension_semantics=("arbitrary",)),
    )(num_prop, w_gate, w1, b1r, W2, b2, Wo, bor)
    return out, loss[0, 0]


# speedup vs baseline: 4.3117x; 1.0074x over previous
import jax
import jax.numpy as jnp
from jax.experimental import pallas as pl
from jax.experimental.pallas import tpu as pltpu

_N, _D, _E, _H, _M = 8192, 1024, 8, 128, 1024
_BN = 256
_GRID = _N // _BN


def _moe_body(x_ref, wg_ref, w1_ref, b1_ref, w2_ref, b2_ref, wo_ref, bo_ref,
              out_ref, loss_ref, imp_ref, load_ref):
    pid = pl.program_id(0)
    x = x_ref[...]
    logits = jnp.dot(x, wg_ref[...], preferred_element_type=jnp.float32)

    ids = jax.lax.broadcasted_iota(jnp.int32, (_BN, _E), 1)
    l1 = jnp.max(logits, axis=1, keepdims=True)
    i1 = jnp.min(jnp.where(logits == l1, ids, _E), axis=1, keepdims=True)
    masked = jnp.where(ids == i1, jnp.float32(-1e30), logits)
    l2 = jnp.max(masked, axis=1, keepdims=True)
    i2 = jnp.min(jnp.where(masked == l2, ids, _E), axis=1, keepdims=True)

    e21 = jnp.exp(l2 - l1)
    g1 = 1.0 / (1.0 + e21)
    g2 = e21 / (1.0 + e21)

    oh1 = (ids == i1).astype(jnp.float32)
    oh2 = (ids == i2).astype(jnp.float32)
    gates = oh1 * g1 + oh2 * g2

    @pl.when(pid == 0)
    def _():
        imp_ref[...] = jnp.zeros_like(imp_ref)
        load_ref[...] = jnp.zeros_like(load_ref)

    imp_ref[...] += jnp.sum(gates, axis=0, keepdims=True)
    load_ref[...] += jnp.sum((gates > 0).astype(jnp.float32), axis=0,
                             keepdims=True)

    xb = x.astype(jnp.bfloat16)
    h_all = jnp.maximum(
        jnp.dot(xb, w1_ref[...], preferred_element_type=jnp.float32)
        + b1_ref[...], 0.0)
    hb_all = h_all.astype(jnp.bfloat16)

    acc = jnp.zeros((_BN, _M), dtype=jnp.float32)
    for e in range(_E):
        h = hb_all[:, e * _H:(e + 1) * _H]
        z = jnp.dot(h, w2_ref[e], preferred_element_type=jnp.float32) \
            + b2_ref[e, :][None, :]
        mx = jnp.max(z, axis=1, keepdims=True)
        ez = jnp.exp(z - mx)
        s = jnp.sum(ez, axis=1, keepdims=True)
        ge = gates[:, e:e + 1]
        acc = acc + ez * (ge / s)

    out_ref[...] = jnp.dot(acc, wo_ref[...],
                           preferred_element_type=jnp.float32) + bo_ref[...]

    @pl.when(pid == _GRID - 1)
    def _():
        def cv2(v):
            m = jnp.sum(v) / _E
            d = v - m
            var = jnp.sum(d * d) / (_E - 1)
            return var / (m * m + 1e-10)
        loss = (cv2(imp_ref[...]) + cv2(load_ref[...])) * 1e-2
        loss_ref[...] = jnp.full((1, 1), loss, dtype=jnp.float32)


def kernel(num_prop, cat_prop, w_gate, W1, b1, W2, b2, Wo, bo):
    w1 = jnp.transpose(W1, (1, 0, 2)).reshape(_D, _E * _H).astype(jnp.bfloat16)
    w2 = W2.astype(jnp.bfloat16)
    b1r = b1.reshape(1, _E * _H)
    bor = bo.reshape(1, 2)

    out, loss = pl.pallas_call(
        _moe_body,
        grid=(_GRID,),
        in_specs=[
            pl.BlockSpec((_BN, _D), lambda i: (i, 0)),
            pl.BlockSpec((_D, _E), lambda i: (0, 0)),
            pl.BlockSpec((_D, _E * _H), lambda i: (0, 0)),
            pl.BlockSpec((1, _E * _H), lambda i: (0, 0)),
            pl.BlockSpec((_E, _H, _M), lambda i: (0, 0, 0)),
            pl.BlockSpec((_E, _M), lambda i: (0, 0)),
            pl.BlockSpec((_M, 2), lambda i: (0, 0)),
            pl.BlockSpec((1, 2), lambda i: (0, 0)),
        ],
        out_specs=[
            pl.BlockSpec((_BN, 2), lambda i: (i, 0)),
            pl.BlockSpec((1, 1), lambda i: (0, 0)),
        ],
        out_shape=[
            jax.ShapeDtypeStruct((_N, 2), jnp.float32),
            jax.ShapeDtypeStruct((1, 1), jnp.float32),
        ],
        scratch_shapes=[
            pltpu.VMEM((1, _E), jnp.float32),
            pltpu.VMEM((1, _E), jnp.float32),
        ],
        compiler_params=pltpu.CompilerParams(
            dimension_semantics=("arbitrary",)),
    )(num_prop, w_gate, w1, b1r, w2, b2, Wo, bor)
    return out, loss[0, 0]
